# baseline (speedup 1.0000x reference)
import functools

import jax
import jax.numpy as jnp
from jax import lax
from jax.experimental import pallas as pl
from jax.experimental.pallas import tpu as pltpu
from jax.experimental.pallas import tpu_sc as plsc

B = 4096
L_BAG = 20
D_CUIS = 32
TSTRIDE = 33
N_CUIS = 1000
D_EX = 48

NC = 2
NS = 16
LANES = 16
NW = NC * NS
BPW = B // NW
GROUPS = BPW // LANES


def _sc_body(cuis_hbm, ids_hbm, mask_hbm, sm_hbm, rb_hbm, pk_hbm,
             wsm_hbm, wrb_hbm, wpk_hbm, out_hbm,
             table_v, ids_v, mask_v, sm_v, rb_v, pk_v,
             wsm_v, wrb_v, wpk_v, obuf, sem):
    wid = lax.axis_index("s") * NC + lax.axis_index("c")
    base = wid * BPW

    copies = [
        pltpu.async_copy(cuis_hbm, table_v, sem),
        pltpu.async_copy(ids_hbm.at[pl.ds(base * L_BAG, BPW * L_BAG)],
                         ids_v, sem),
        pltpu.async_copy(mask_hbm.at[pl.ds(base * L_BAG, BPW * L_BAG)],
                         mask_v, sem),
        pltpu.async_copy(sm_hbm.at[pl.ds(base, BPW)], sm_v, sem),
        pltpu.async_copy(rb_hbm.at[pl.ds(base, BPW)], rb_v, sem),
        pltpu.async_copy(pk_hbm.at[pl.ds(base, BPW)], pk_v, sem),
        pltpu.async_copy(wsm_hbm, wsm_v, sem),
        pltpu.async_copy(wrb_hbm, wrb_v, sem),
        pltpu.async_copy(wpk_hbm, wpk_v, sem),
    ]
    for c in copies:
        c.wait()

    def splat(val):
        return jnp.full((LANES,), val, jnp.int32)

    iota16 = lax.iota(jnp.int32, LANES)

    def group_body(g, carry):
        r0 = g * LANES

        rows_l = (r0 + iota16) * L_BAG

        msum = plsc.load_gather(mask_v, [rows_l])
        for l in range(1, L_BAG):
            msum = msum + plsc.load_gather(mask_v, [rows_l + l])
        pos = msum > 0.0
        denom = jnp.maximum(msum, 1e-9)

        for dh in range(2):
            accs = [jnp.zeros((LANES,), jnp.float32) for _ in range(16)]
            prev = None
            for l in range(L_BAG):
                rid_s = plsc.load_gather(ids_v, [rows_l + l]) * TSTRIDE
                m = plsc.load_gather(mask_v, [rows_l + l])
                loads = [plsc.load_gather(table_v,
                                          [rid_s + splat(dh * 16 + d)])
                         for d in range(16)]
                if prev is not None:
                    pm, pls_ = prev
                    for d in range(16):
                        accs[d] = accs[d] + pm * pls_[d]
                prev = (m, loads)
            pm, pls_ = prev
            for d in range(16):
                accs[d] = accs[d] + pm * pls_[d]
            for d in range(16):
                col = dh * 16 + d
                obuf[pl.ds(col * BPW + r0, LANES)] = jnp.where(
                    pos, accs[d] / denom, 0.0)

        sid = sm_v[pl.ds(r0, LANES)] * 5
        for d in range(4):
            obuf[pl.ds((32 + d) * BPW + r0, LANES)] = plsc.load_gather(
                wsm_v, [sid + splat(d)])
        rid2 = rb_v[pl.ds(r0, LANES)] * 9
        for d in range(8):
            obuf[pl.ds((36 + d) * BPW + r0, LANES)] = plsc.load_gather(
                wrb_v, [rid2 + splat(d)])
        pid = pk_v[pl.ds(r0, LANES)] * 5
        for d in range(4):
            obuf[pl.ds((44 + d) * BPW + r0, LANES)] = plsc.load_gather(
                wpk_v, [pid + splat(d)])
        return carry

    lax.fori_loop(0, GROUPS, group_body, 0)
    pltpu.sync_copy(obuf, out_hbm.at[pl.ds(base * D_EX, BPW * D_EX)])


@functools.cache
def _sc_extract_fn():
    return functools.partial(
        pl.kernel,
        out_type=jax.ShapeDtypeStruct((B * D_EX,), jnp.float32),
        mesh=plsc.VectorSubcoreMesh(core_axis_name="c", subcore_axis_name="s",
                                    num_cores=NC, num_subcores=NS),
        compiler_params=pltpu.CompilerParams(needs_layout_passes=False,
                                             disable_bounds_checks=True),
        scratch_types=[
            pltpu.VMEM((N_CUIS * TSTRIDE,), jnp.float32),
            pltpu.VMEM((BPW * L_BAG,), jnp.int32),
            pltpu.VMEM((BPW * L_BAG,), jnp.float32),
            pltpu.VMEM((BPW,), jnp.int32),
            pltpu.VMEM((BPW,), jnp.int32),
            pltpu.VMEM((BPW,), jnp.int32),
            pltpu.VMEM((40,), jnp.float32),
            pltpu.VMEM((72,), jnp.float32),
            pltpu.VMEM((40,), jnp.float32),
            pltpu.VMEM((BPW * D_EX,), jnp.float32),
            pltpu.SemaphoreType.DMA,
        ],
    )(_sc_body)


def _tc1_body(bert_ref, w1b_ref, hpre_ref):
    hpre_ref[...] = jnp.dot(bert_ref[...], w1b_ref[...],
                            preferred_element_type=jnp.float32)


def _tc1_bert(bert, w1b, block_b=512):
    nblk = B // block_b
    return pl.pallas_call(
        _tc1_body,
        grid=(nblk,),
        in_specs=[
            pl.BlockSpec((block_b, 768), lambda i: (i, 0)),
            pl.BlockSpec((768, 512), lambda i: (0, 0)),
        ],
        out_specs=pl.BlockSpec((block_b, 512), lambda i: (i, 0)),
        out_shape=jax.ShapeDtypeStruct((B, 512), jnp.float32),
        compiler_params=pltpu.CompilerParams(
            dimension_semantics=("arbitrary",)),
    )(bert, w1b)


def _tc2_body(num_ref, ord_ref, hpre_ref, ex_ref, w1num_ref, w1ord_ref,
              w1ex_ref, b1_ref, w2_ref, b2_ref, out_ref):
    h = hpre_ref[...] + jnp.dot(num_ref[...], w1num_ref[...],
                                preferred_element_type=jnp.float32)
    h = h + jnp.dot(ord_ref[...], w1ord_ref[...],
                    preferred_element_type=jnp.float32)
    hx = [lax.dot_general(ex_ref[j], w1ex_ref[...],
                          (((0,), (0,)), ((), ())),
                          preferred_element_type=jnp.float32)
          for j in range(4)]
    h = h + jnp.concatenate(hx, axis=0)
    h = jnp.maximum(h + b1_ref[...], 0.0)
    out = jnp.dot(h, w2_ref[...], preferred_element_type=jnp.float32)
    out = out + b2_ref[...]
    nrm = jnp.sqrt(jnp.sum(out * out, axis=1, keepdims=True))
    out_ref[...] = out / jnp.maximum(nrm, 1e-12)


def _tc_mlp(num, ordf, hpre, ex_t, w1num, w1ord, w1ex, b1, w2, b2,
            block_b=512):
    nblk = B // block_b
    full = lambda shape: pl.BlockSpec(shape, lambda i: (0,) * len(shape))
    wpb = block_b // BPW
    return pl.pallas_call(
        _tc2_body,
        grid=(nblk,),
        in_specs=[
            pl.BlockSpec((block_b, 16), lambda i: (i, 0)),
            pl.BlockSpec((block_b, 8), lambda i: (i, 0)),
            pl.BlockSpec((block_b, 512), lambda i: (i, 0)),
            pl.BlockSpec((wpb, D_EX, BPW), lambda i: (i, 0, 0)),
            full((16, 512)),
            full((8, 512)),
            full((D_EX, 512)),
            full((1, 512)),
            full((512, 512)),
            full((1, 512)),
        ],
        out_specs=pl.BlockSpec((block_b, 512), lambda i: (i, 0)),
        out_shape=jax.ShapeDtypeStruct((B, 512), jnp.float32),
        compiler_params=pltpu.CompilerParams(
            dimension_semantics=("arbitrary",)),
    )(num, ordf, hpre, ex_t, w1num, w1ord, w1ex, b1, w2, b2)


def kernel(smoking_area_id, rambience_id, parking_lot_id, cuisine_ids,
           cuisine_mask, numeric_feats, ordinal_feats, bert_emb,
           W_smoking, W_ramb, W_park, W_cuisine, W1, b1, W2, b2):
    sm = smoking_area_id.astype(jnp.int32)
    rb = rambience_id.astype(jnp.int32)
    pk = parking_lot_id.astype(jnp.int32)

    ids_t = cuisine_ids.astype(jnp.int32).reshape(-1)
    mask_t = cuisine_mask.reshape(-1)

    wcu = jnp.pad(W_cuisine, ((0, 0), (0, TSTRIDE - D_CUIS))).reshape(-1)
    wsm = jnp.zeros((8, 5), jnp.float32).at[:3, :4].set(W_smoking).reshape(-1)
    wrb = jnp.zeros((8, 9), jnp.float32).at[:5, :8].set(W_ramb).reshape(-1)
    wpk = jnp.zeros((8, 5), jnp.float32).at[:4, :4].set(W_park).reshape(-1)

    ex = _sc_extract_fn()(wcu, ids_t, mask_t, sm, rb, pk, wsm, wrb, wpk)
    ex_t = ex.reshape(NW, D_EX, BPW)

    w1ex = jnp.concatenate([W1[24:56], W1[824:840]], axis=0)

    hpre = _tc1_bert(bert_emb, W1[56:824])

    return _tc_mlp(numeric_feats, ordinal_feats, hpre, ex_t,
                   W1[0:16], W1[16:24], w1ex,
                   b1.reshape(1, 512), W2, b2.reshape(1, 512))


# speedup vs baseline: 1.1532x; 1.1532x over previous
import functools

import jax
import jax.numpy as jnp
from jax import lax
from jax.experimental import pallas as pl
from jax.experimental.pallas import tpu as pltpu
from jax.experimental.pallas import tpu_sc as plsc

B = 4096
L_BAG = 20
D_CUIS = 32
TSTRIDE = 33
N_CUIS = 1000
D_EX = 48

NC = 2
NS = 16
LANES = 16
NW = NC * NS
BPW = B // NW
GROUPS = BPW // LANES


def _sc_body(cuis_hbm, ids_hbm, mask_hbm, sm_hbm, rb_hbm, pk_hbm,
             wsm_hbm, wrb_hbm, wpk_hbm, out_hbm,
             table_v, ids_v, mask_v, sm_v, rb_v, pk_v,
             wsm_v, wrb_v, wpk_v, obuf, sem):
    wid = lax.axis_index("s") * NC + lax.axis_index("c")
    base = wid * BPW

    copies = [
        pltpu.async_copy(cuis_hbm, table_v, sem),
        pltpu.async_copy(ids_hbm.at[pl.ds(base * L_BAG, BPW * L_BAG)],
                         ids_v, sem),
        pltpu.async_copy(mask_hbm.at[pl.ds(base * L_BAG, BPW * L_BAG)],
                         mask_v, sem),
        pltpu.async_copy(sm_hbm.at[pl.ds(base, BPW)], sm_v, sem),
        pltpu.async_copy(rb_hbm.at[pl.ds(base, BPW)], rb_v, sem),
        pltpu.async_copy(pk_hbm.at[pl.ds(base, BPW)], pk_v, sem),
        pltpu.async_copy(wsm_hbm, wsm_v, sem),
        pltpu.async_copy(wrb_hbm, wrb_v, sem),
        pltpu.async_copy(wpk_hbm, wpk_v, sem),
    ]
    for c in copies:
        c.wait()

    def splat(val):
        return jnp.full((LANES,), val, jnp.int32)

    iota16 = lax.iota(jnp.int32, LANES)

    def group_body(g, carry):
        r0 = g * LANES

        msum = mask_v[pl.ds(r0, LANES)]
        for l in range(1, L_BAG):
            msum = msum + mask_v[pl.ds(l * BPW + r0, LANES)]
        pos = msum > 0.0
        denom = jnp.maximum(msum, 1e-9)

        for dh in range(2):
            accs = [jnp.zeros((LANES,), jnp.float32) for _ in range(16)]
            prev = None
            for l in range(L_BAG):
                rid_s = ids_v[pl.ds(l * BPW + r0, LANES)] * TSTRIDE
                m = mask_v[pl.ds(l * BPW + r0, LANES)]
                loads = [plsc.load_gather(table_v,
                                          [rid_s + splat(dh * 16 + d)])
                         for d in range(16)]
                if prev is not None:
                    pm, pls_ = prev
                    for d in range(16):
                        accs[d] = accs[d] + pm * pls_[d]
                prev = (m, loads)
            pm, pls_ = prev
            for d in range(16):
                accs[d] = accs[d] + pm * pls_[d]
            for d in range(16):
                col = dh * 16 + d
                obuf[pl.ds(col * BPW + r0, LANES)] = jnp.where(
                    pos, accs[d] / denom, 0.0)

        sid = sm_v[pl.ds(r0, LANES)] * 5
        for d in range(4):
            obuf[pl.ds((32 + d) * BPW + r0, LANES)] = plsc.load_gather(
                wsm_v, [sid + splat(d)])
        rid2 = rb_v[pl.ds(r0, LANES)] * 9
        for d in range(8):
            obuf[pl.ds((36 + d) * BPW + r0, LANES)] = plsc.load_gather(
                wrb_v, [rid2 + splat(d)])
        pid = pk_v[pl.ds(r0, LANES)] * 5
        for d in range(4):
            obuf[pl.ds((44 + d) * BPW + r0, LANES)] = plsc.load_gather(
                wpk_v, [pid + splat(d)])
        return carry

    lax.fori_loop(0, GROUPS, group_body, 0)
    pltpu.sync_copy(obuf, out_hbm.at[pl.ds(base * D_EX, BPW * D_EX)])


@functools.cache
def _sc_extract_fn():
    return functools.partial(
        pl.kernel,
        out_type=jax.ShapeDtypeStruct((B * D_EX,), jnp.float32),
        mesh=plsc.VectorSubcoreMesh(core_axis_name="c", subcore_axis_name="s",
                                    num_cores=NC, num_subcores=NS),
        compiler_params=pltpu.CompilerParams(needs_layout_passes=False,
                                             disable_bounds_checks=True),
        scratch_types=[
            pltpu.VMEM((N_CUIS * TSTRIDE,), jnp.float32),
            pltpu.VMEM((BPW * L_BAG,), jnp.int32),
            pltpu.VMEM((BPW * L_BAG,), jnp.float32),
            pltpu.VMEM((BPW,), jnp.int32),
            pltpu.VMEM((BPW,), jnp.int32),
            pltpu.VMEM((BPW,), jnp.int32),
            pltpu.VMEM((40,), jnp.float32),
            pltpu.VMEM((72,), jnp.float32),
            pltpu.VMEM((40,), jnp.float32),
            pltpu.VMEM((BPW * D_EX,), jnp.float32),
            pltpu.SemaphoreType.DMA,
        ],
    )(_sc_body)


def _tc_body(num_ref, ord_ref, bert_ref, ex_ref, w1num_ref, w1ord_ref,
             w1b_ref, w1ex_ref, b1_ref, w2_ref, b2_ref, out_ref):
    bf = jnp.bfloat16
    h = jnp.dot(bert_ref[...].astype(bf), w1b_ref[...].astype(bf),
                preferred_element_type=jnp.float32)
    h = h + jnp.dot(num_ref[...], w1num_ref[...],
                    preferred_element_type=jnp.float32)
    h = h + jnp.dot(ord_ref[...], w1ord_ref[...],
                    preferred_element_type=jnp.float32)
    hx = [lax.dot_general(ex_ref[j], w1ex_ref[...],
                          (((0,), (0,)), ((), ())),
                          preferred_element_type=jnp.float32)
          for j in range(4)]
    h = h + jnp.concatenate(hx, axis=0)
    h = jnp.maximum(h + b1_ref[...], 0.0)
    out = jnp.dot(h.astype(bf), w2_ref[...].astype(bf),
                  preferred_element_type=jnp.float32)
    out = out + b2_ref[...]
    nrm = jnp.sqrt(jnp.sum(out * out, axis=1, keepdims=True))
    out_ref[...] = out / jnp.maximum(nrm, 1e-12)


def _tc_mlp(num, ordf, bert, ex_t, w1num, w1ord, w1b, w1ex, b1, w2, b2,
            block_b=512):
    nblk = B // block_b
    full = lambda shape: pl.BlockSpec(shape, lambda i: (0,) * len(shape))
    wpb = block_b // BPW
    return pl.pallas_call(
        _tc_body,
        grid=(nblk,),
        in_specs=[
            pl.BlockSpec((block_b, 16), lambda i: (i, 0)),
            pl.BlockSpec((block_b, 8), lambda i: (i, 0)),
            pl.BlockSpec((block_b, 768), lambda i: (i, 0)),
            pl.BlockSpec((wpb, D_EX, BPW), lambda i: (i, 0, 0)),
            full((16, 512)),
            full((8, 512)),
            full((768, 512)),
            full((D_EX, 512)),
            full((1, 512)),
            full((512, 512)),
            full((1, 512)),
        ],
        out_specs=pl.BlockSpec((block_b, 512), lambda i: (i, 0)),
        out_shape=jax.ShapeDtypeStruct((B, 512), jnp.float32),
        compiler_params=pltpu.CompilerParams(
            dimension_semantics=("arbitrary",)),
    )(num, ordf, bert, ex_t, w1num, w1ord, w1b, w1ex, b1, w2, b2)


def kernel(smoking_area_id, rambience_id, parking_lot_id, cuisine_ids,
           cuisine_mask, numeric_feats, ordinal_feats, bert_emb,
           W_smoking, W_ramb, W_park, W_cuisine, W1, b1, W2, b2):
    sm = smoking_area_id.astype(jnp.int32)
    rb = rambience_id.astype(jnp.int32)
    pk = parking_lot_id.astype(jnp.int32)

    ids_t = (cuisine_ids.astype(jnp.int32)
             .reshape(NW, BPW, L_BAG).transpose(0, 2, 1).reshape(-1))
    mask_t = (cuisine_mask
              .reshape(NW, BPW, L_BAG).transpose(0, 2, 1).reshape(-1))

    wcu = jnp.pad(W_cuisine, ((0, 0), (0, TSTRIDE - D_CUIS))).reshape(-1)
    wsm = jnp.zeros((8, 5), jnp.float32).at[:3, :4].set(W_smoking).reshape(-1)
    wrb = jnp.zeros((8, 9), jnp.float32).at[:5, :8].set(W_ramb).reshape(-1)
    wpk = jnp.zeros((8, 5), jnp.float32).at[:4, :4].set(W_park).reshape(-1)

    ex = _sc_extract_fn()(wcu, ids_t, mask_t, sm, rb, pk, wsm, wrb, wpk)
    ex_t = ex.reshape(NW, D_EX, BPW)

    w1ex = jnp.concatenate([W1[24:56], W1[824:840]], axis=0)

    return _tc_mlp(numeric_feats, ordinal_feats, bert_emb, ex_t,
                   W1[0:16], W1[16:24], W1[56:824], w1ex,
                   b1.reshape(1, 512), W2, b2.reshape(1, 512))


# speedup vs baseline: 1.1895x; 1.0314x over previous
import functools

import jax
import jax.numpy as jnp
from jax import lax
from jax.experimental import pallas as pl
from jax.experimental.pallas import tpu as pltpu
from jax.experimental.pallas import tpu_sc as plsc

B = 4096
L_BAG = 20
D_CUIS = 32
TSTRIDE = 33
N_CUIS = 1000
D_EX = 48

NC = 2
NS = 16
LANES = 16
NW = NC * NS
BPW = B // NW
GROUPS = BPW // LANES


def _sc_body(cuis_hbm, ids_hbm, mask_hbm, sm_hbm, rb_hbm, pk_hbm,
             wsm_hbm, wrb_hbm, wpk_hbm, out_hbm,
             table_v, ids_v, mask_v, sm_v, rb_v, pk_v,
             wsm_v, wrb_v, wpk_v, obuf, sem):
    wid = lax.axis_index("s") * NC + lax.axis_index("c")
    base = wid * BPW

    copies = [
        pltpu.async_copy(cuis_hbm, table_v, sem),
        pltpu.async_copy(ids_hbm.at[pl.ds(base * L_BAG, BPW * L_BAG)],
                         ids_v, sem),
        pltpu.async_copy(mask_hbm.at[pl.ds(base * L_BAG, BPW * L_BAG)],
                         mask_v, sem),
        pltpu.async_copy(sm_hbm.at[pl.ds(base, BPW)], sm_v, sem),
        pltpu.async_copy(rb_hbm.at[pl.ds(base, BPW)], rb_v, sem),
        pltpu.async_copy(pk_hbm.at[pl.ds(base, BPW)], pk_v, sem),
        pltpu.async_copy(wsm_hbm, wsm_v, sem),
        pltpu.async_copy(wrb_hbm, wrb_v, sem),
        pltpu.async_copy(wpk_hbm, wpk_v, sem),
    ]
    for c in copies:
        c.wait()

    def splat(val):
        return jnp.full((LANES,), val, jnp.int32)

    iota16 = lax.iota(jnp.int32, LANES)

    def group_body(g, carry):
        r0 = g * LANES

        msum = mask_v[pl.ds(r0, LANES)]
        for l in range(1, L_BAG):
            msum = msum + mask_v[pl.ds(l * BPW + r0, LANES)]
        pos = msum > 0.0
        denom = jnp.maximum(msum, 1e-9)

        for dh in range(2):
            accs = [jnp.zeros((LANES,), jnp.float32) for _ in range(16)]
            prev = None
            for l in range(L_BAG):
                rid_s = ids_v[pl.ds(l * BPW + r0, LANES)] * TSTRIDE
                m = mask_v[pl.ds(l * BPW + r0, LANES)]
                loads = [plsc.load_gather(table_v,
                                          [rid_s + splat(dh * 16 + d)])
                         for d in range(16)]
                if prev is not None:
                    pm, pls_ = prev
                    for d in range(16):
                        accs[d] = accs[d] + pm * pls_[d]
                prev = (m, loads)
            pm, pls_ = prev
            for d in range(16):
                accs[d] = accs[d] + pm * pls_[d]
            for d in range(16):
                col = dh * 16 + d
                obuf[pl.ds(col * BPW + r0, LANES)] = jnp.where(
                    pos, accs[d] / denom, 0.0)

        sid = sm_v[pl.ds(r0, LANES)] * 5
        for d in range(4):
            obuf[pl.ds((32 + d) * BPW + r0, LANES)] = plsc.load_gather(
                wsm_v, [sid + splat(d)])
        rid2 = rb_v[pl.ds(r0, LANES)] * 9
        for d in range(8):
            obuf[pl.ds((36 + d) * BPW + r0, LANES)] = plsc.load_gather(
                wrb_v, [rid2 + splat(d)])
        pid = pk_v[pl.ds(r0, LANES)] * 5
        for d in range(4):
            obuf[pl.ds((44 + d) * BPW + r0, LANES)] = plsc.load_gather(
                wpk_v, [pid + splat(d)])
        return carry

    lax.fori_loop(0, GROUPS, group_body, 0)
    pltpu.sync_copy(obuf, out_hbm.at[pl.ds(base * D_EX, BPW * D_EX)])


@functools.cache
def _sc_extract_fn():
    return functools.partial(
        pl.kernel,
        out_type=jax.ShapeDtypeStruct((B * D_EX,), jnp.float32),
        mesh=plsc.VectorSubcoreMesh(core_axis_name="c", subcore_axis_name="s",
                                    num_cores=NC, num_subcores=NS),
        compiler_params=pltpu.CompilerParams(needs_layout_passes=False,
                                             disable_bounds_checks=True),
        scratch_types=[
            pltpu.VMEM((N_CUIS * TSTRIDE,), jnp.float32),
            pltpu.VMEM((BPW * L_BAG,), jnp.int32),
            pltpu.VMEM((BPW * L_BAG,), jnp.float32),
            pltpu.VMEM((BPW,), jnp.int32),
            pltpu.VMEM((BPW,), jnp.int32),
            pltpu.VMEM((BPW,), jnp.int32),
            pltpu.VMEM((40,), jnp.float32),
            pltpu.VMEM((72,), jnp.float32),
            pltpu.VMEM((40,), jnp.float32),
            pltpu.VMEM((BPW * D_EX,), jnp.float32),
            pltpu.SemaphoreType.DMA,
        ],
    )(_sc_body)


def _tc_body(num_ref, ord_ref, bert_ref, ex_ref, w1num_ref, w1ord_ref,
             w1b_ref, w1ex_ref, b1_ref, w2_ref, b2_ref, out_ref, *,
             wpb):
    bf = jnp.bfloat16
    h = jnp.dot(bert_ref[...].astype(bf), w1b_ref[...].astype(bf),
                preferred_element_type=jnp.float32)
    h = h + jnp.dot(num_ref[...], w1num_ref[...],
                    preferred_element_type=jnp.float32)
    h = h + jnp.dot(ord_ref[...], w1ord_ref[...],
                    preferred_element_type=jnp.float32)
    hx = [lax.dot_general(ex_ref[j], w1ex_ref[...],
                          (((0,), (0,)), ((), ())),
                          preferred_element_type=jnp.float32)
          for j in range(wpb)]
    h = h + jnp.concatenate(hx, axis=0)
    h = jnp.maximum(h + b1_ref[...], 0.0)
    out = jnp.dot(h.astype(bf), w2_ref[...].astype(bf),
                  preferred_element_type=jnp.float32)
    out = out + b2_ref[...]
    nrm = jnp.sqrt(jnp.sum(out * out, axis=1, keepdims=True))
    out_ref[...] = out / jnp.maximum(nrm, 1e-12)


def _tc_mlp(num, ordf, bert, ex_t, w1num, w1ord, w1b, w1ex, b1, w2, b2,
            block_b=1024):
    nblk = B // block_b
    full = lambda shape: pl.BlockSpec(shape, lambda i: (0,) * len(shape))
    wpb = block_b // BPW
    return pl.pallas_call(
        functools.partial(_tc_body, wpb=wpb),
        grid=(nblk,),
        in_specs=[
            pl.BlockSpec((block_b, 16), lambda i: (i, 0)),
            pl.BlockSpec((block_b, 8), lambda i: (i, 0)),
            pl.BlockSpec((block_b, 768), lambda i: (i, 0)),
            pl.BlockSpec((wpb, D_EX, BPW), lambda i: (i, 0, 0)),
            full((16, 512)),
            full((8, 512)),
            full((768, 512)),
            full((D_EX, 512)),
            full((1, 512)),
            full((512, 512)),
            full((1, 512)),
        ],
        out_specs=pl.BlockSpec((block_b, 512), lambda i: (i, 0)),
        out_shape=jax.ShapeDtypeStruct((B, 512), jnp.float32),
        compiler_params=pltpu.CompilerParams(
            dimension_semantics=("arbitrary",)),
    )(num, ordf, bert, ex_t, w1num, w1ord, w1b, w1ex, b1, w2, b2)


def kernel(smoking_area_id, rambience_id, parking_lot_id, cuisine_ids,
           cuisine_mask, numeric_feats, ordinal_feats, bert_emb,
           W_smoking, W_ramb, W_park, W_cuisine, W1, b1, W2, b2):
    sm = smoking_area_id.astype(jnp.int32)
    rb = rambience_id.astype(jnp.int32)
    pk = parking_lot_id.astype(jnp.int32)

    ids_t = (cuisine_ids.astype(jnp.int32)
             .reshape(NW, BPW, L_BAG).transpose(0, 2, 1).reshape(-1))
    mask_t = (cuisine_mask
              .reshape(NW, BPW, L_BAG).transpose(0, 2, 1).reshape(-1))

    wcu = jnp.pad(W_cuisine, ((0, 0), (0, TSTRIDE - D_CUIS))).reshape(-1)
    wsm = jnp.zeros((8, 5), jnp.float32).at[:3, :4].set(W_smoking).reshape(-1)
    wrb = jnp.zeros((8, 9), jnp.float32).at[:5, :8].set(W_ramb).reshape(-1)
    wpk = jnp.zeros((8, 5), jnp.float32).at[:4, :4].set(W_park).reshape(-1)

    ex = _sc_extract_fn()(wcu, ids_t, mask_t, sm, rb, pk, wsm, wrb, wpk)
    ex_t = ex.reshape(NW, D_EX, BPW)

    w1ex = jnp.concatenate([W1[24:56], W1[824:840]], axis=0)

    return _tc_mlp(numeric_feats, ordinal_feats, bert_emb, ex_t,
                   W1[0:16], W1[16:24], W1[56:824], w1ex,
                   b1.reshape(1, 512), W2, b2.reshape(1, 512))
